# baseline (speedup 1.0000x reference)
import functools

import numpy as np
import jax
import jax.numpy as jnp
from jax import lax
from jax.experimental import pallas as pl
from jax.experimental.pallas import tpu as pltpu
from jax.experimental.pallas import tpu_sc as plsc

MAXL = 8192
BATCH = 2
D = 768
NC, NS, LANES = 2, 16, 16
NW = NC * NS
L_PER_W = MAXL // NW
PC = 16
CHUNK = BATCH * PC
NCHUNK = L_PER_W // PC
NGRP = D // LANES


def _make_pos_factors():
    d = np.arange(D)
    w = 10000.0 ** (-2.0 * (d // 2) / D)
    l0 = (np.arange(NW * NCHUNK) * PC)[:, None]
    even = (d % 2 == 0)
    u = np.where(even, np.sin(l0 * w), np.cos(l0 * w))
    v = np.where(even, np.cos(l0 * w), -np.sin(l0 * w))
    uv = np.stack([u, v], axis=1).reshape(NW, NCHUNK, 2, D).astype(np.float32)

    t = np.arange(PC)[:, None]
    cs = np.stack([np.cos(t * w), np.sin(t * w)], axis=0
                  ).astype(np.float32)
    return uv, cs


_UV, _CS = _make_pos_factors()


@functools.partial(
    pl.kernel,
    out_type=jax.ShapeDtypeStruct((MAXL, BATCH, D), jnp.float32),
    mesh=plsc.VectorSubcoreMesh(core_axis_name="c", subcore_axis_name="s"),
    scratch_types=(
        [pltpu.VMEM((NCHUNK, CHUNK), jnp.int32),
         pltpu.VMEM((2, PC, D), jnp.float32)]
        + [pltpu.VMEM((CHUNK, D), jnp.float32) for _ in range(2)]
        + [pltpu.VMEM((PC, BATCH, D), jnp.float32) for _ in range(2)]
        + [pltpu.VMEM((2, D), jnp.float32) for _ in range(2)]
        + [pltpu.SemaphoreType.DMA for _ in range(4)]
    ),
)
def _emb_kernel(x_hbm, uv_hbm, cs_hbm, table_hbm, out_hbm, idx_v, cs_v,
                g0, g1, st0, st1, uv0, uv1,
                gsem0, gsem1, osem0, osem1):
    gbuf = (g0, g1)
    stbuf = (st0, st1)
    uvb = (uv0, uv1)
    gsem = (gsem0, gsem1)
    osem = (osem0, osem1)

    wid = lax.axis_index("s") * NC + lax.axis_index("c")
    lbase = wid * L_PER_W

    def start(j):
        s = j % 2
        g = pltpu.async_copy(table_hbm.at[idx_v.at[j]], gbuf[s], gsem[s])
        p = pltpu.async_copy(uv_hbm.at[wid, j], uvb[s], gsem[s])
        return (g, p)

    pltpu.sync_copy(x_hbm.at[wid], idx_v)

    descs = [None] * NCHUNK
    odescs = [None] * NCHUNK
    descs[0] = start(0)
    descs[1] = start(1)

    pltpu.sync_copy(cs_hbm, cs_v)

    for j in range(NCHUNK):
        s = j % 2
        for dsc in descs[j]:
            dsc.wait()
        if j - 2 >= 0:
            odescs[j - 2].wait()

        rs = gbuf[s]
        st = stbuf[s]
        uvs = uvb[s]

        def grp_body(grp, carry):
            sl = pl.ds(grp * LANES, LANES)
            u = uvs[0, sl]
            v = uvs[1, sl]

            @plsc.parallel_loop(0, PC, unroll=4)
            def _(t):
                pv = u * cs_v[0, t, sl] + v * cs_v[1, t, sl]
                a = rs[2 * t, sl]
                b = rs[2 * t + 1, sl]
                st[t, 0, sl] = a + a + pv
                st[t, 1, sl] = b + b + pv

            return carry

        lax.fori_loop(0, NGRP, grp_body, 0)

        l0 = lbase + j * PC
        odescs[j] = pltpu.async_copy(st, out_hbm.at[pl.ds(l0, PC)], osem[s])
        if j + 2 < NCHUNK:
            descs[j + 2] = start(j + 2)

    odescs[NCHUNK - 2].wait()
    odescs[NCHUNK - 1].wait()


def kernel(x, table):
    xi = x.astype(jnp.int32).reshape(NW, NCHUNK, CHUNK)
    return _emb_kernel(xi, jnp.asarray(_UV), jnp.asarray(_CS), table)
